# baseline (speedup 1.0000x reference)
import functools

import jax
import jax.numpy as jnp
from jax import lax
from jax.experimental import pallas as pl
from jax.experimental.pallas import tpu as pltpu
from jax.experimental.pallas import tpu_sc as plsc

_RB = 8000
_CHUNK = 128


@functools.cache
def _tc_copy(m, d, dtype):
    def body(x_ref, o_ref):
        o_ref[...] = x_ref[...]

    return pl.pallas_call(
        body,
        grid=(-(-m // _RB),),
        in_specs=[pl.BlockSpec((_RB, d), lambda i: (i, 0))],
        out_specs=pl.BlockSpec((_RB, d), lambda i: (i, 0)),
        out_shape=jax.ShapeDtypeStruct((m, d), dtype),
    )


@functools.cache
def _sc_scatter(m, d, b, dtype):
    try:
        info = plsc.get_sparse_core_info()
        nc, ns, nl = info.num_cores, info.num_subcores, info.num_lanes
    except ValueError:
        nc, ns, nl = 2, 16, 16
    nw = nc * ns
    tile_rows = -(-m // nw)
    c = _CHUNK
    nch = (b + c) // c
    mesh = plsc.VectorSubcoreMesh(
        core_axis_name="c", subcore_axis_name="s",
        num_cores=nc, num_subcores=ns)

    @functools.partial(
        pl.kernel,
        mesh=mesh,
        out_type=(),
        compiler_params=pltpu.CompilerParams(
            needs_layout_passes=False, use_tc_tiling_on_sc=False),
        scratch_types=[
            pltpu.VMEM((b,), jnp.int32),
            pltpu.VMEM((tile_rows,), jnp.int32),
            pltpu.VMEM((nch, c), jnp.int32),
            pltpu.VMEM((nch, c), jnp.int32),
            pltpu.VMEM((c, d), dtype),
            pltpu.SemaphoreType.DMA,
            pltpu.SemaphoreType.DMA,
            pltpu.SemaphoreType.DMA,
        ],
    )
    def scatter(out_ref, idx_ref, val_ref, idx_v, tag, wrow, wpos,
                rows_buf, isem, sem_g, sem_s):
        wid = lax.axis_index("s") * nc + lax.axis_index("c")
        lo = wid * tile_rows
        iota = lax.iota(jnp.int32, nl)
        pltpu.async_copy(idx_ref, idx_v, isem).wait()

        def in_range(q):
            v = idx_v[pl.ds(q * nl, nl)]
            vloc = v - lo
            msk = (vloc >= 0) & (vloc < tile_rows)
            return v, jnp.where(msk, vloc, 0), msk, q * nl + iota

        def pass_a(q, carry):
            _, safe, msk, pos = in_range(q)
            plsc.store_scatter(tag, [safe], pos, mask=msk)
            return carry

        lax.fori_loop(0, b // nl, pass_a, 0, unroll=8)

        def pass_b(q, cnt):
            v, safe, msk, pos = in_range(q)
            t = plsc.load_gather(tag, [safe], mask=msk)
            win = msk & (t == pos)
            incl = plsc.cumsum(win.astype(jnp.int32))
            slot = jnp.where(win, cnt + incl - 1, 0)
            plsc.store_scatter(wrow, [slot // c, slot % c], v, mask=win)
            plsc.store_scatter(wpos, [slot // c, slot % c], pos, mask=win)
            return cnt + jnp.max(incl)

        cnt = lax.fori_loop(0, b // nl, pass_b, jnp.int32(0), unroll=8)

        @pl.when(cnt > 0)
        def _():
            head = wrow[0, pl.ds(0, nl)]
            headp = wpos[0, pl.ds(0, nl)]
            fr = jnp.max(jnp.where(iota == 0, head, -1))
            fp = jnp.max(jnp.where(iota == 0, headp, -1))
            for k in range(c // nl):
                slots = cnt + k * nl + iota
                plsc.store_scatter(wrow, [slots // c, slots % c],
                                   jnp.full((nl,), fr, jnp.int32))
                plsc.store_scatter(wpos, [slots // c, slots % c],
                                   jnp.full((nl,), fp, jnp.int32))

            def chunk(cc, carry):
                pltpu.async_copy(val_ref.at[wpos.at[cc]], rows_buf, sem_g).wait()
                pltpu.async_copy(rows_buf, out_ref.at[wrow.at[cc]], sem_s).wait()
                return carry

            lax.fori_loop(0, (cnt + c - 1) // c, chunk, 0)

    return scatter


def kernel(mem, idx, val):
    m, d = mem.shape
    b = idx.shape[0]
    out = _tc_copy(m, d, mem.dtype)(mem)
    ref = jax.new_ref(out)
    _sc_scatter(m, d, b, mem.dtype)(ref, idx, val)
    return ref[...]
